# baseline (speedup 1.0000x reference)
import functools

import jax
import jax.numpy as jnp
from jax import lax
from jax.experimental import pallas as pl
from jax.experimental.pallas import tpu as pltpu
from jax.experimental.pallas import tpu_sc as plsc

N_USERS = 9000
N_NODES = 10000
D = 128
E = 320000
N_SEG = 500
N_NUDGE = 500

NC = 2
NS = 16
NW = NC * NS
EPW = E // NW
C = 80
NCHUNK = EPW // C
N_PAD = 10240
RPT = N_PAD // NS

_MESH = plsc.VectorSubcoreMesh(core_axis_name="c", subcore_axis_name="s")


def _sc_a_pass(h, ei, w, zA):
  def body(h_hbm, ei_hbm, w_hbm, zA_hbm, outA, accA,
           srcAll, *bufs_flat):
    cid = lax.axis_index("c")
    sid = lax.axis_index("s")
    wid = sid * NC + cid
    rslice = pl.ds(sid * RPT, RPT)
    bufs = (bufs_flat[0:7], bufs_flat[7:14], bufs_flat[14:21])

    ebase = pl.multiple_of(wid * EPW, 8)
    dbase = pl.multiple_of(E + wid * EPW, 8)
    pltpu.sync_copy(zA_hbm.at[rslice], accA.at[rslice])
    pltpu.sync_copy(ei_hbm.at[pl.ds(ebase, EPW)], srcAll)
    plsc.subcore_barrier()

    def issue_stream(j, ph):
      rows, dstv, wv, gsem, _, dsem, wsem = bufs[ph]
      cbase = pl.multiple_of(j * C, 8)
      pltpu.async_copy(h_hbm.at[srcAll.at[pl.ds(cbase, C)]], rows, gsem)
      pltpu.async_copy(ei_hbm.at[pl.ds(dbase + cbase, C)], dstv, dsem)
      pltpu.async_copy(w_hbm.at[pl.ds(ebase + cbase, C)],
                       wv.at[pl.ds(0, C)], wsem)

    def drain_scatter(ph):
      rows, dstv, _, _, ssem, _, _ = bufs[ph]
      pltpu.make_async_copy(rows, accA.at[dstv], ssem).wait()

    def step(j, ph, do_drain, do_issue):
      rows, dstv, wv, gsem, ssem, dsem, wsem = bufs[ph]
      nph = (ph + 1) % 3
      cbase = pl.multiple_of(j * C, 8)
      if do_drain:
        drain_scatter(nph)
      if do_issue:
        issue_stream(j + 1, nph)
      pltpu.make_async_copy(
          h_hbm.at[srcAll.at[pl.ds(cbase, C)]], rows, gsem).wait()
      pltpu.make_async_copy(w_hbm.at[pl.ds(ebase + cbase, C)],
                            wv.at[pl.ds(0, C)], wsem).wait()

      def grp(g, c2):
        wg = wv[pl.ds(g * 16, 16)]
        for r2 in range(16):
          wspl = jnp.broadcast_to(wg[r2], (16,))
          e = g * 16 + r2
          for f in range(D // 16):
            sl = pl.ds(f * 16, 16)
            rows[e, sl] = rows[e, sl] * wspl
        return c2

      lax.fori_loop(0, C // 16, grp, 0)
      pltpu.make_async_copy(
          ei_hbm.at[pl.ds(dbase + cbase, C)], dstv, dsem).wait()
      pltpu.async_copy(rows, accA.at[dstv], ssem, add=True)

    issue_stream(0, 0)
    step(0, 0, False, True)
    step(1, 1, False, True)

    def triple(p, carry):
      j0 = 2 + 3 * p
      step(j0, 2, True, True)
      step(j0 + 1, 0, True, True)
      step(j0 + 2, 1, True, True)
      return carry

    lax.fori_loop(0, (NCHUNK - 5) // 3, triple, 0)
    step(NCHUNK - 3, 2, True, True)
    step(NCHUNK - 2, 0, True, True)
    step(NCHUNK - 1, 1, True, False)
    drain_scatter(0)
    drain_scatter(1)

    plsc.subcore_barrier()
    pltpu.sync_copy(accA.at[rslice], outA.at[cid, rslice])

  per_phase = [
      pltpu.VMEM((C, D), jnp.float32),
      pltpu.VMEM((C,), jnp.int32),
      pltpu.VMEM((128,), jnp.float32),
      pltpu.SemaphoreType.DMA,
      pltpu.SemaphoreType.DMA,
      pltpu.SemaphoreType.DMA,
      pltpu.SemaphoreType.DMA,
  ]
  fn = pl.kernel(
      body, mesh=_MESH,
      out_type=jax.ShapeDtypeStruct((NC, N_PAD, D), jnp.float32),
      scratch_types=[pltpu.VMEM_SHARED((N_PAD, D), jnp.float32),
                     pltpu.VMEM((EPW,), jnp.int32)]
                    + per_phase * 3)
  return fn(h, ei, w, zA)


def _sc_sw_pass(ei, w, zA):
  def body(ei_hbm, w_hbm, zA_hbm, outS, accW, dstAll, *bufs_flat):
    cid = lax.axis_index("c")
    sid = lax.axis_index("s")
    wid = sid * NC + cid
    rslice = pl.ds(sid * RPT, RPT)
    bufs = (bufs_flat[0:4], bufs_flat[4:8], bufs_flat[8:12])
    z16 = jnp.zeros((16,), jnp.float32)

    ebase = pl.multiple_of(wid * EPW, 8)
    dbase = pl.multiple_of(E + wid * EPW, 8)
    pltpu.sync_copy(zA_hbm.at[rslice], accW.at[rslice])
    pltpu.sync_copy(ei_hbm.at[pl.ds(dbase, EPW)], dstAll)

    def zrow(e, c2):
      for b in bufs:
        for f in range(D // 16):
          b[0][e, pl.ds(f * 16, 16)] = z16
      return c2
    lax.fori_loop(0, C, zrow, 0)
    plsc.subcore_barrier()

    def issue_w(j, ph):
      _, wv, _, wsem = bufs[ph]
      cbase = pl.multiple_of(j * C, 8)
      pltpu.async_copy(w_hbm.at[pl.ds(ebase + cbase, C)],
                       wv.at[pl.ds(0, C)], wsem)

    def drain_scatter(j, ph):
      swr, _, ssem, _ = bufs[ph]
      cbase = pl.multiple_of(j * C, 8)
      pltpu.make_async_copy(
          swr, accW.at[dstAll.at[pl.ds(cbase, C)]], ssem).wait()

    def step(j, ph, do_drain, do_issue):
      swr, wv, ssem, wsem = bufs[ph]
      nph = (ph + 1) % 3
      cbase = pl.multiple_of(j * C, 8)
      if do_drain:
        drain_scatter(j - 2, nph)
      if do_issue:
        issue_w(j + 1, nph)
      pltpu.make_async_copy(w_hbm.at[pl.ds(ebase + cbase, C)],
                            wv.at[pl.ds(0, C)], wsem).wait()

      def grp(g, c2):
        wg = wv[pl.ds(g * 16, 16)]
        for r2 in range(16):
          wspl = jnp.broadcast_to(wg[r2], (16,))
          e = g * 16 + r2
          swr[e, pl.ds(0, 16)] = wspl
        return c2

      lax.fori_loop(0, C // 16, grp, 0)
      pltpu.async_copy(
          swr, accW.at[dstAll.at[pl.ds(cbase, C)]], ssem, add=True)

    issue_w(0, 0)
    step(0, 0, False, True)
    step(1, 1, False, True)

    def triple(p, carry):
      j0 = 2 + 3 * p
      step(j0, 2, True, True)
      step(j0 + 1, 0, True, True)
      step(j0 + 2, 1, True, True)
      return carry

    lax.fori_loop(0, (NCHUNK - 5) // 3, triple, 0)
    step(NCHUNK - 3, 2, True, True)
    step(NCHUNK - 2, 0, True, True)
    step(NCHUNK - 1, 1, True, False)
    drain_scatter(NCHUNK - 2, 0)
    drain_scatter(NCHUNK - 1, 1)

    plsc.subcore_barrier()
    pltpu.sync_copy(accW.at[rslice], outS.at[cid, rslice])

  per_phase = [
      pltpu.VMEM((C, D), jnp.float32),
      pltpu.VMEM((128,), jnp.float32),
      pltpu.SemaphoreType.DMA,
      pltpu.SemaphoreType.DMA,
  ]
  fn = pl.kernel(
      body, mesh=_MESH,
      out_type=jax.ShapeDtypeStruct((NC, N_PAD, D), jnp.float32),
      scratch_types=[
          pltpu.VMEM_SHARED((N_PAD, D), jnp.float32),
          pltpu.VMEM((EPW,), jnp.int32)]
          + per_phase * 3)
  return fn(ei, w, zA)


_BU = 1000
_BN = 400
_BP = 600


def _enc_body(x_ref, w1_ref, b1_ref, w2_ref, b2_ref, emb_ref, o_ref):
  i = pl.program_id(0)

  @pl.when(i < N_USERS // _BU)
  def _mlp():
    h = jnp.dot(x_ref[...], w1_ref[...], preferred_element_type=jnp.float32)
    h = jnp.maximum(h + b1_ref[...], 0.0)
    o_ref[...] = (jnp.dot(h, w2_ref[...], preferred_element_type=jnp.float32)
                  + b2_ref[...])

  @pl.when(i == N_USERS // _BU)
  def _emb():
    o_ref[...] = emb_ref[...]


def _encoder(x, W1, b1, W2, b2, emb):
  return pl.pallas_call(
      _enc_body,
      grid=(N_NODES // _BU,),
      in_specs=[pl.BlockSpec((_BU, D), lambda i: (i, 0)),
                pl.BlockSpec((D, D), lambda i: (0, 0)),
                pl.BlockSpec((1, D), lambda i: (0, 0)),
                pl.BlockSpec((D, D), lambda i: (0, 0)),
                pl.BlockSpec((1, D), lambda i: (0, 0)),
                pl.BlockSpec((N_SEG + N_NUDGE, D), lambda i: (0, 0))],
      out_specs=pl.BlockSpec((_BU, D), lambda i: (i, 0)),
      out_shape=jax.ShapeDtypeStruct((N_NODES, D), jnp.float32),
  )(x, W1, b1.reshape(1, D), W2, b2.reshape(1, D), emb)


def _conv_body(h_ref, a_ref, s_ref, w1_ref, w2_ref, bb_ref, o_ref):
  h = h_ref[...]
  A = a_ref[0] + a_ref[1]
  sw = s_ref[0][:, 0:1] + s_ref[1][:, 0:1]
  t = (jnp.dot(sw * h + A, w1_ref[...], preferred_element_type=jnp.float32)
       + jnp.dot(h * A, w2_ref[...], preferred_element_type=jnp.float32)
       + sw * bb_ref[...])
  y = jnp.where(t >= 0.0, t, 0.01 * t)
  nrm = jnp.sqrt(jnp.sum(y * y, axis=1, keepdims=True))
  o_ref[...] = y / jnp.maximum(nrm, 1e-12)


def _conv_update(h, A_part, Sw_part, W1, W2, bb):
  return pl.pallas_call(
      _conv_body,
      grid=(N_NODES // _BN,),
      in_specs=[pl.BlockSpec((_BN, D), lambda i: (i, 0)),
                pl.BlockSpec((NC, _BN, D), lambda i: (0, i, 0)),
                pl.BlockSpec((NC, _BN, D), lambda i: (0, i, 0)),
                pl.BlockSpec((D, D), lambda i: (0, 0)),
                pl.BlockSpec((D, D), lambda i: (0, 0)),
                pl.BlockSpec((1, D), lambda i: (0, 0))],
      out_specs=pl.BlockSpec((_BN, D), lambda i: (i, 0)),
      out_shape=jax.ShapeDtypeStruct((N_NODES, D), jnp.float32),
  )(h, A_part, Sw_part, W1, W2, bb.reshape(1, D))


_NP512 = 512


def _pred_body(h0_ref, h1_ref, h2_ref, sw1_ref, sb1_ref, sw2_ref, sb2_ref,
               nw1_ref, nb1_ref, nw2_ref, nb2_ref, o1_ref, o2_ref):
  u0 = h0_ref[...]
  u1 = h1_ref[...]
  u2 = h2_ref[...]

  def head(w1_ref, b1_ref, w2_ref, b2_ref, o_ref):
    t = (jnp.dot(u0, w1_ref[0], preferred_element_type=jnp.float32)
         + jnp.dot(u1, w1_ref[1], preferred_element_type=jnp.float32)
         + jnp.dot(u2, w1_ref[2], preferred_element_type=jnp.float32))
    hh = jnp.maximum(t + b1_ref[...], 0.0)
    o_ref[...] = (jnp.dot(hh, w2_ref[...],
                          preferred_element_type=jnp.float32) + b2_ref[...])

  head(sw1_ref, sb1_ref, sw2_ref, sb2_ref, o1_ref)
  head(nw1_ref, nb1_ref, nw2_ref, nb2_ref, o2_ref)


def _predictors(h0, h1, h2, sp_W1, sp_b1, sp_W2, sp_b2,
                np_W1, np_b1, np_W2, np_b2):
  hspec = pl.BlockSpec((_BP, D), lambda i: (i, 0))
  pw = ((0, 0), (0, _NP512 - N_SEG))
  o1, o2 = pl.pallas_call(
      _pred_body,
      grid=(N_USERS // _BP,),
      in_specs=[hspec, hspec, hspec,
                pl.BlockSpec((3, D, D), lambda i: (0, 0, 0)),
                pl.BlockSpec((1, D), lambda i: (0, 0)),
                pl.BlockSpec((D, _NP512), lambda i: (0, 0)),
                pl.BlockSpec((1, _NP512), lambda i: (0, 0)),
                pl.BlockSpec((3, D, D), lambda i: (0, 0, 0)),
                pl.BlockSpec((1, D), lambda i: (0, 0)),
                pl.BlockSpec((D, _NP512), lambda i: (0, 0)),
                pl.BlockSpec((1, _NP512), lambda i: (0, 0))],
      out_specs=[pl.BlockSpec((_BP, _NP512), lambda i: (i, 0)),
                 pl.BlockSpec((_BP, _NP512), lambda i: (i, 0))],
      out_shape=[jax.ShapeDtypeStruct((N_USERS, _NP512), jnp.float32),
                 jax.ShapeDtypeStruct((N_USERS, _NP512), jnp.float32)],
  )(h0, h1, h2,
    sp_W1.reshape(3, D, D), sp_b1.reshape(1, D),
    jnp.pad(sp_W2, pw), jnp.pad(sp_b2.reshape(1, N_SEG), pw),
    np_W1.reshape(3, D, D), np_b1.reshape(1, D),
    jnp.pad(np_W2, pw), jnp.pad(np_b2.reshape(1, N_NUDGE), pw))
  return o1[:, :N_SEG], o2[:, :N_NUDGE]


def kernel(x, edge_index, edge_weight, fe_W1, fe_b1, fe_W2, fe_b2,
           seg_emb, nud_emb,
           c1_W1, c1_b1, c1_W2, c1_b2, c2_W1, c2_b1, c2_W2, c2_b2,
           sp_W1, sp_b1, sp_W2, sp_b2, np_W1, np_b1, np_W2, np_b2):
  ei = edge_index.reshape(2 * E)

  emb = jnp.concatenate([seg_emb, nud_emb], axis=0)
  h0 = _encoder(x, fe_W1, fe_b1, fe_W2, fe_b2, emb)

  zA = jnp.zeros((N_PAD, D), jnp.float32)

  S1 = _sc_sw_pass(ei, edge_weight, zA)
  A1 = _sc_a_pass(h0, ei, edge_weight, zA)
  h1 = _conv_update(h0, A1, S1, c1_W1, c1_W2, c1_b1 + c1_b2)
  A2 = _sc_a_pass(h1, ei, edge_weight, zA)
  h2 = _conv_update(h1, A2, S1, c2_W1, c2_W2, c2_b1 + c2_b2)

  return _predictors(h0, h1, h2, sp_W1, sp_b1, sp_W2, sp_b2,
                     np_W1, np_b1, np_W2, np_b2)


# speedup vs baseline: 1.2748x; 1.2748x over previous
import functools

import jax
import jax.numpy as jnp
from jax import lax
from jax.experimental import pallas as pl
from jax.experimental.pallas import tpu as pltpu
from jax.experimental.pallas import tpu_sc as plsc

N_USERS = 9000
N_NODES = 10000
D = 128
E = 320000
N_SEG = 500
N_NUDGE = 500

NC = 2
NS = 16
NW = NC * NS
EPW = E // NW
C = 80
NCHUNK = EPW // C
N_PAD = 10240
RPT = N_PAD // NS

_MESH = plsc.VectorSubcoreMesh(core_axis_name="c", subcore_axis_name="s")


def _sc_a_pass(h, ei, w, zA):
  def body(h_hbm, ei_hbm, w_hbm, zA_hbm, outA, accA,
           srcAll, *bufs_flat):
    cid = lax.axis_index("c")
    sid = lax.axis_index("s")
    wid = sid * NC + cid
    rslice = pl.ds(sid * RPT, RPT)
    bufs = (bufs_flat[0:7], bufs_flat[7:14], bufs_flat[14:21])

    ebase = pl.multiple_of(wid * EPW, 8)
    dbase = pl.multiple_of(E + wid * EPW, 8)
    pltpu.sync_copy(zA_hbm.at[rslice], accA.at[rslice])
    pltpu.sync_copy(ei_hbm.at[pl.ds(ebase, EPW)], srcAll)
    plsc.subcore_barrier()

    def issue_stream(j, ph):
      rows, dstv, wv, gsem, _, dsem, wsem = bufs[ph]
      cbase = pl.multiple_of(j * C, 8)
      pltpu.async_copy(h_hbm.at[srcAll.at[pl.ds(cbase, C)]], rows, gsem)
      pltpu.async_copy(ei_hbm.at[pl.ds(dbase + cbase, C)], dstv, dsem)
      pltpu.async_copy(w_hbm.at[pl.ds(ebase + cbase, C)],
                       wv.at[pl.ds(0, C)], wsem)

    def drain_scatter(ph):
      rows, dstv, _, _, ssem, _, _ = bufs[ph]
      pltpu.make_async_copy(rows, accA.at[dstv], ssem).wait()

    def step(j, ph, do_drain, do_issue):
      rows, dstv, wv, gsem, ssem, dsem, wsem = bufs[ph]
      nph = (ph + 1) % 3
      cbase = pl.multiple_of(j * C, 8)
      if do_drain:
        drain_scatter(nph)
      if do_issue:
        issue_stream(j + 1, nph)
      pltpu.make_async_copy(
          h_hbm.at[srcAll.at[pl.ds(cbase, C)]], rows, gsem).wait()
      pltpu.make_async_copy(w_hbm.at[pl.ds(ebase + cbase, C)],
                            wv.at[pl.ds(0, C)], wsem).wait()

      def grp(g, c2):
        wg = wv[pl.ds(g * 16, 16)]
        for r2 in range(16):
          wspl = jnp.broadcast_to(wg[r2], (16,))
          e = g * 16 + r2
          for f in range(D // 16):
            sl = pl.ds(f * 16, 16)
            rows[e, sl] = rows[e, sl] * wspl
        return c2

      lax.fori_loop(0, C // 16, grp, 0)
      pltpu.make_async_copy(
          ei_hbm.at[pl.ds(dbase + cbase, C)], dstv, dsem).wait()
      pltpu.async_copy(rows, accA.at[dstv], ssem, add=True)

    issue_stream(0, 0)
    step(0, 0, False, True)
    step(1, 1, False, True)

    def triple(p, carry):
      j0 = 2 + 3 * p
      step(j0, 2, True, True)
      step(j0 + 1, 0, True, True)
      step(j0 + 2, 1, True, True)
      return carry

    lax.fori_loop(0, (NCHUNK - 5) // 3, triple, 0)
    step(NCHUNK - 3, 2, True, True)
    step(NCHUNK - 2, 0, True, True)
    step(NCHUNK - 1, 1, True, False)
    drain_scatter(0)
    drain_scatter(1)

    plsc.subcore_barrier()
    pltpu.sync_copy(accA.at[rslice], outA.at[cid, rslice])

  per_phase = [
      pltpu.VMEM((C, D), jnp.float32),
      pltpu.VMEM((C,), jnp.int32),
      pltpu.VMEM((128,), jnp.float32),
      pltpu.SemaphoreType.DMA,
      pltpu.SemaphoreType.DMA,
      pltpu.SemaphoreType.DMA,
      pltpu.SemaphoreType.DMA,
  ]
  fn = pl.kernel(
      body, mesh=_MESH,
      out_type=jax.ShapeDtypeStruct((NC, N_PAD, D), jnp.float32),
      scratch_types=[pltpu.VMEM_SHARED((N_PAD, D), jnp.float32),
                     pltpu.VMEM((EPW,), jnp.int32)]
                    + per_phase * 3)
  return fn(h, ei, w, zA)


def _sc_sw_pass(ei, w, zA):
  def body(ei_hbm, w_hbm, zA_hbm, outS, accW, dstAll, *bufs_flat):
    cid = lax.axis_index("c")
    sid = lax.axis_index("s")
    wid = sid * NC + cid
    rslice = pl.ds(sid * RPT, RPT)
    bufs = (bufs_flat[0:4], bufs_flat[4:8], bufs_flat[8:12])
    z16 = jnp.zeros((16,), jnp.float32)

    ebase = pl.multiple_of(wid * EPW, 8)
    dbase = pl.multiple_of(E + wid * EPW, 8)
    pltpu.sync_copy(zA_hbm.at[rslice], accW.at[rslice])
    pltpu.sync_copy(ei_hbm.at[pl.ds(dbase, EPW)], dstAll)

    def zrow(e, c2):
      for b in bufs:
        for f in range(D // 16):
          b[0][e, pl.ds(f * 16, 16)] = z16
      return c2
    lax.fori_loop(0, C, zrow, 0)
    plsc.subcore_barrier()

    def issue_w(j, ph):
      _, wv, _, wsem = bufs[ph]
      cbase = pl.multiple_of(j * C, 8)
      pltpu.async_copy(w_hbm.at[pl.ds(ebase + cbase, C)],
                       wv.at[pl.ds(0, C)], wsem)

    def drain_scatter(j, ph):
      swr, _, ssem, _ = bufs[ph]
      cbase = pl.multiple_of(j * C, 8)
      pltpu.make_async_copy(
          swr, accW.at[dstAll.at[pl.ds(cbase, C)]], ssem).wait()

    def step(j, ph, do_drain, do_issue):
      swr, wv, ssem, wsem = bufs[ph]
      nph = (ph + 1) % 3
      cbase = pl.multiple_of(j * C, 8)
      if do_drain:
        drain_scatter(j - 2, nph)
      if do_issue:
        issue_w(j + 1, nph)
      pltpu.make_async_copy(w_hbm.at[pl.ds(ebase + cbase, C)],
                            wv.at[pl.ds(0, C)], wsem).wait()

      def grp(g, c2):
        wg = wv[pl.ds(g * 16, 16)]
        for r2 in range(16):
          wspl = jnp.broadcast_to(wg[r2], (16,))
          e = g * 16 + r2
          swr[e, pl.ds(0, 16)] = wspl
        return c2

      lax.fori_loop(0, C // 16, grp, 0)
      pltpu.async_copy(
          swr, accW.at[dstAll.at[pl.ds(cbase, C)]], ssem, add=True)

    issue_w(0, 0)
    step(0, 0, False, True)
    step(1, 1, False, True)

    def triple(p, carry):
      j0 = 2 + 3 * p
      step(j0, 2, True, True)
      step(j0 + 1, 0, True, True)
      step(j0 + 2, 1, True, True)
      return carry

    lax.fori_loop(0, (NCHUNK - 5) // 3, triple, 0)
    step(NCHUNK - 3, 2, True, True)
    step(NCHUNK - 2, 0, True, True)
    step(NCHUNK - 1, 1, True, False)
    drain_scatter(NCHUNK - 2, 0)
    drain_scatter(NCHUNK - 1, 1)

    plsc.subcore_barrier()
    pltpu.sync_copy(accW.at[rslice], outS.at[cid, rslice])

  per_phase = [
      pltpu.VMEM((C, D), jnp.float32),
      pltpu.VMEM((128,), jnp.float32),
      pltpu.SemaphoreType.DMA,
      pltpu.SemaphoreType.DMA,
  ]
  fn = pl.kernel(
      body, mesh=_MESH,
      out_type=jax.ShapeDtypeStruct((NC, N_PAD, D), jnp.float32),
      scratch_types=[
          pltpu.VMEM_SHARED((N_PAD, D), jnp.float32),
          pltpu.VMEM((EPW,), jnp.int32)]
          + per_phase * 3)
  return fn(ei, w, zA)


_BU = 1000
_BN = 400
_BP = 600


def _enc_body(x_ref, w1_ref, b1_ref, w2_ref, b2_ref, emb_ref, o_ref):
  i = pl.program_id(0)

  @pl.when(i < N_USERS // _BU)
  def _mlp():
    h = jnp.dot(x_ref[...], w1_ref[...], preferred_element_type=jnp.float32)
    h = jnp.maximum(h + b1_ref[...], 0.0)
    o_ref[...] = (jnp.dot(h, w2_ref[...], preferred_element_type=jnp.float32)
                  + b2_ref[...])

  @pl.when(i == N_USERS // _BU)
  def _emb():
    o_ref[...] = emb_ref[...]


def _encoder(x, W1, b1, W2, b2, emb):
  return pl.pallas_call(
      _enc_body,
      grid=(N_NODES // _BU,),
      in_specs=[pl.BlockSpec((_BU, D), lambda i: (i, 0)),
                pl.BlockSpec((D, D), lambda i: (0, 0)),
                pl.BlockSpec((1, D), lambda i: (0, 0)),
                pl.BlockSpec((D, D), lambda i: (0, 0)),
                pl.BlockSpec((1, D), lambda i: (0, 0)),
                pl.BlockSpec((N_SEG + N_NUDGE, D), lambda i: (0, 0))],
      out_specs=pl.BlockSpec((_BU, D), lambda i: (i, 0)),
      out_shape=jax.ShapeDtypeStruct((N_NODES, D), jnp.float32),
  )(x, W1, b1.reshape(1, D), W2, b2.reshape(1, D), emb)


def _conv_body(h_ref, a_ref, s_ref, w1_ref, w2_ref, bb_ref, o_ref):
  h = h_ref[...]
  A = a_ref[0] + a_ref[1]
  sw = s_ref[0][:, 0:1] + s_ref[1][:, 0:1]
  t = (jnp.dot(sw * h + A, w1_ref[...], preferred_element_type=jnp.float32)
       + jnp.dot(h * A, w2_ref[...], preferred_element_type=jnp.float32)
       + sw * bb_ref[...])
  y = jnp.where(t >= 0.0, t, 0.01 * t)
  nrm = jnp.sqrt(jnp.sum(y * y, axis=1, keepdims=True))
  o_ref[...] = y / jnp.maximum(nrm, 1e-12)


def _conv_update(h, A_part, Sw_part, W1, W2, bb):
  return pl.pallas_call(
      _conv_body,
      grid=(N_NODES // _BN,),
      in_specs=[pl.BlockSpec((_BN, D), lambda i: (i, 0)),
                pl.BlockSpec((NC, _BN, D), lambda i: (0, i, 0)),
                pl.BlockSpec((NC, _BN, D), lambda i: (0, i, 0)),
                pl.BlockSpec((D, D), lambda i: (0, 0)),
                pl.BlockSpec((D, D), lambda i: (0, 0)),
                pl.BlockSpec((1, D), lambda i: (0, 0))],
      out_specs=pl.BlockSpec((_BN, D), lambda i: (i, 0)),
      out_shape=jax.ShapeDtypeStruct((N_NODES, D), jnp.float32),
  )(h, A_part, Sw_part, W1, W2, bb.reshape(1, D))


def _pred_partial_body(h0_ref, h1_ref, sw_ref, nw_ref, os_ref, on_ref):
  u0 = h0_ref[...]
  u1 = h1_ref[...]
  os_ref[...] = (jnp.dot(u0, sw_ref[0], preferred_element_type=jnp.float32)
                 + jnp.dot(u1, sw_ref[1], preferred_element_type=jnp.float32))
  on_ref[...] = (jnp.dot(u0, nw_ref[0], preferred_element_type=jnp.float32)
                 + jnp.dot(u1, nw_ref[1], preferred_element_type=jnp.float32))


def _pred_partial(h0, h1, sp_W1, np_W1):
  hspec = pl.BlockSpec((_BP, D), lambda i: (i, 0))
  wspec = pl.BlockSpec((2, D, D), lambda i: (0, 0, 0))
  return pl.pallas_call(
      _pred_partial_body,
      grid=(N_USERS // _BP,),
      in_specs=[hspec, hspec, wspec, wspec],
      out_specs=[hspec, hspec],
      out_shape=[jax.ShapeDtypeStruct((N_USERS, D), jnp.float32),
                 jax.ShapeDtypeStruct((N_USERS, D), jnp.float32)],
  )(h0, h1, sp_W1.reshape(3, D, D)[0:2], np_W1.reshape(3, D, D)[0:2])


def _pred_body(ps_ref, pn_ref, h2_ref, sw1_ref, sb1_ref, sw2_ref, sb2_ref,
               nw1_ref, nb1_ref, nw2_ref, nb2_ref, o1_ref, o2_ref):
  u2 = h2_ref[...]

  def head(p_ref, w1_ref, b1_ref, w2_ref, b2_ref, o_ref):
    t = p_ref[...] + jnp.dot(u2, w1_ref[...],
                             preferred_element_type=jnp.float32)
    hh = jnp.maximum(t + b1_ref[...], 0.0)
    o_ref[...] = (jnp.dot(hh, w2_ref[...],
                          preferred_element_type=jnp.float32) + b2_ref[...])

  head(ps_ref, sw1_ref, sb1_ref, sw2_ref, sb2_ref, o1_ref)
  head(pn_ref, nw1_ref, nb1_ref, nw2_ref, nb2_ref, o2_ref)


def _predictors(ps, pn, h2, sp_W1, sp_b1, sp_W2, sp_b2,
                np_W1, np_b1, np_W2, np_b2):
  hspec = pl.BlockSpec((_BP, D), lambda i: (i, 0))
  return pl.pallas_call(
      _pred_body,
      grid=(N_USERS // _BP,),
      in_specs=[hspec, hspec, hspec,
                pl.BlockSpec((D, D), lambda i: (0, 0)),
                pl.BlockSpec((1, D), lambda i: (0, 0)),
                pl.BlockSpec((D, N_SEG), lambda i: (0, 0)),
                pl.BlockSpec((1, N_SEG), lambda i: (0, 0)),
                pl.BlockSpec((D, D), lambda i: (0, 0)),
                pl.BlockSpec((1, D), lambda i: (0, 0)),
                pl.BlockSpec((D, N_NUDGE), lambda i: (0, 0)),
                pl.BlockSpec((1, N_NUDGE), lambda i: (0, 0))],
      out_specs=[pl.BlockSpec((_BP, N_SEG), lambda i: (i, 0)),
                 pl.BlockSpec((_BP, N_NUDGE), lambda i: (i, 0))],
      out_shape=[jax.ShapeDtypeStruct((N_USERS, N_SEG), jnp.float32),
                 jax.ShapeDtypeStruct((N_USERS, N_NUDGE), jnp.float32)],
  )(ps, pn, h2,
    sp_W1.reshape(3, D, D)[2], sp_b1.reshape(1, D),
    sp_W2, sp_b2.reshape(1, N_SEG),
    np_W1.reshape(3, D, D)[2], np_b1.reshape(1, D),
    np_W2, np_b2.reshape(1, N_NUDGE))


def kernel(x, edge_index, edge_weight, fe_W1, fe_b1, fe_W2, fe_b2,
           seg_emb, nud_emb,
           c1_W1, c1_b1, c1_W2, c1_b2, c2_W1, c2_b1, c2_W2, c2_b2,
           sp_W1, sp_b1, sp_W2, sp_b2, np_W1, np_b1, np_W2, np_b2):
  ei = edge_index.reshape(2 * E)

  emb = jnp.concatenate([seg_emb, nud_emb], axis=0)
  h0 = _encoder(x, fe_W1, fe_b1, fe_W2, fe_b2, emb)

  zA = jnp.zeros((N_PAD, D), jnp.float32)

  S1 = _sc_sw_pass(ei, edge_weight, zA)
  A1 = _sc_a_pass(h0, ei, edge_weight, zA)
  h1 = _conv_update(h0, A1, S1, c1_W1, c1_W2, c1_b1 + c1_b2)
  A2 = _sc_a_pass(h1, ei, edge_weight, zA)
  ps, pn = _pred_partial(h0, h1, sp_W1, np_W1)
  h2 = _conv_update(h1, A2, S1, c2_W1, c2_W2, c2_b1 + c2_b2)

  return _predictors(ps, pn, h2, sp_W1, sp_b1, sp_W2, sp_b2,
                     np_W1, np_b1, np_W2, np_b2)
